# Initial kernel scaffold; baseline (speedup 1.0000x reference)
import jax
import jax.numpy as jnp
from jax.experimental import pallas as pl


def kernel(x, w1, b1, w2, b2):
    raise NotImplementedError("write your pallas kernel here")


# speedup vs baseline: 1.5540x; 1.5540x over previous
import jax
import jax.numpy as jnp
from jax.experimental import pallas as pl
from jax.experimental.pallas import tpu as pltpu


def _gate_kernel(x_ref, w1t_ref, b1_ref, w2t_ref, b2_ref, o_ref, *, inv_s):
    xt = x_ref[0]
    s_sum = jnp.sum(xt, axis=1, keepdims=True)
    s_max = jnp.max(xt, axis=1, keepdims=True)
    pools = jnp.concatenate([s_sum * inv_s, s_max], axis=1)
    h = jnp.dot(w1t_ref[...], pools,
                preferred_element_type=jnp.float32) + b1_ref[...]
    h = jnp.maximum(h, 0.0)
    att2 = jnp.dot(w2t_ref[...], h,
                   preferred_element_type=jnp.float32) + b2_ref[...]
    att = att2[:, 0:1] + att2[:, 1:2]
    scale = jax.nn.sigmoid(att)
    o_ref[0] = (xt * scale).astype(o_ref.dtype)


def kernel(x, w1, b1, w2, b2):
    B, C, D, H, W = x.shape
    S = D * H * W
    x3 = x.reshape(B, C, S)
    itemsize = jnp.dtype(x.dtype).itemsize

    w1t = w1.astype(jnp.float32).T
    w2t = w2.astype(jnp.float32).T
    b1c = b1.astype(jnp.float32).reshape(-1, 1)
    b2c = b2.astype(jnp.float32).reshape(-1, 1)
    Cr = w1t.shape[0]

    import functools
    body = functools.partial(_gate_kernel, inv_s=1.0 / S)

    cost = pl.CostEstimate(
        flops=4 * B * C * S,
        transcendentals=B * C,
        bytes_accessed=2 * B * C * S * itemsize)

    out = pl.pallas_call(
        body,
        out_shape=jax.ShapeDtypeStruct((B, C, S), x.dtype),
        grid=(B,),
        in_specs=[
            pl.BlockSpec((1, C, S), lambda b: (b, 0, 0)),
            pl.BlockSpec((Cr, C), lambda b: (0, 0)),
            pl.BlockSpec((Cr, 1), lambda b: (0, 0)),
            pl.BlockSpec((C, Cr), lambda b: (0, 0)),
            pl.BlockSpec((C, 1), lambda b: (0, 0)),
        ],
        out_specs=pl.BlockSpec((1, C, S), lambda b: (b, 0, 0)),
        compiler_params=pltpu.CompilerParams(
            dimension_semantics=("parallel",)),
        cost_estimate=cost,
    )(x3, w1t, b1c, w2t, b2c)

    return out.reshape(B, C, D, H, W)
